# baseline (speedup 1.0000x reference)
import functools

import numpy as np
import jax
import jax.numpy as jnp
from jax import lax
from jax.experimental import pallas as pl
from jax.experimental.pallas import tpu as pltpu
from jax.experimental.pallas import tpu_sc as plsc

_VOCAB = 1000
_EMB = 128
_HID = 64
_STEPS = 5
_L = 50
_B = 1024
_NP = 18
_DT = np.float32(0.05 / 5)
_TINY = np.float32(np.finfo(np.float32).tiny)

_ROTS = ((13, 15, 26, 6), (17, 29, 16, 24))


def _np_threefry_block(k0, k1, x0, x1):
    k0 = np.uint32(k0)
    k1 = np.uint32(k1)
    ks = (k0, k1, k0 ^ k1 ^ np.uint32(0x1BD11BDA))
    x0 = np.asarray(x0, np.uint32)
    x1 = np.asarray(x1, np.uint32)
    with np.errstate(over="ignore"):
        x0 = x0 + ks[0]
        x1 = x1 + ks[1]
        for i in range(5):
            for r in _ROTS[i % 2]:
                x0 = x0 + x1
                x1 = (x1 << np.uint32(r)) | (x1 >> np.uint32(32 - r))
                x1 = x1 ^ x0
            x0 = x0 + ks[(i + 1) % 3]
            x1 = x1 + ks[(i + 2) % 3] + np.uint32(i + 1)
    return x0, x1


def _np_fold_in(kp, data):
    w0, w1 = _np_threefry_block(kp[0], kp[1], np.uint32(0), np.uint32(data))
    return (w0, w1)


def _derive_keys():
    base = (np.uint32(0), np.uint32(42))
    kcat = np.zeros((_STEPS, _L, 2), np.uint32)
    kunif = np.zeros((_STEPS, _L, 2), np.uint32)
    for i in range(_STEPS):
        ki = _np_fold_in(base, i)
        for j in range(_L):
            kk = _np_fold_in(ki, j)
            w0, w1 = _np_threefry_block(
                kk[0], kk[1], np.zeros(2, np.uint32), np.arange(2, dtype=np.uint32)
            )
            kcat[i, j] = (w0[0], w1[0])
            kunif[i, j] = (w0[1], w1[1])
    return kcat.view(np.int32), kunif.view(np.int32)


_KCAT, _KUNIF = _derive_keys()


def _tf_bits(k0, k1, m):
    ks2 = k0 ^ k1 ^ jnp.int32(0x1BD11BDA)
    ks = (k0, k1, ks2)
    x0 = jnp.zeros_like(m) + k0
    x1 = m + k1
    for i in range(5):
        for r in _ROTS[i % 2]:
            x0 = x0 + x1
            x1 = lax.shift_left(x1, jnp.int32(r)) | lax.shift_right_logical(
                x1, jnp.int32(32 - r)
            )
            x1 = x1 ^ x0
        x0 = x0 + ks[(i + 1) % 3]
        x1 = x1 + ks[(i + 2) % 3] + jnp.int32(i + 1)
    return x0 ^ x1


def _bits_to_unit(bits):
    fb = lax.shift_right_logical(bits, jnp.int32(9)) | jnp.int32(0x3F800000)
    return lax.bitcast_convert_type(fb, jnp.float32) - jnp.float32(1.0)


def _one_step(cur, cat_bits, ku0, ku1, vio, m_acc, table, w1t, b1, w2t, b2):
    onehot_b = vio == cur
    oh = onehot_b.astype(jnp.float32)
    emb = jnp.dot(oh, table, preferred_element_type=jnp.float32)
    h = jnp.dot(emb, w1t, preferred_element_type=jnp.float32)
    h = jnp.maximum(h + b1, jnp.float32(0.0))
    inten = jnp.dot(h, w2t, preferred_element_type=jnp.float32)
    inten = inten + b2

    ctr = inten - jnp.max(inten, axis=1, keepdims=True)
    e = jnp.exp(ctr)
    s = jnp.sum(e, axis=1, keepdims=True)
    em = jnp.where(onehot_b, jnp.float32(0.0), e)
    s2 = jnp.maximum(jnp.sum(em, axis=1, keepdims=True) / s, jnp.float32(1e-8))
    lsum = jnp.log(s) + jnp.log(s2)
    c = jnp.float32(np.log(np.float32(1e-12)))
    logits = jnp.where(onehot_b, c, jnp.maximum(ctr - lsum, c))

    fb = _bits_to_unit(cat_bits)
    u = jnp.maximum(_TINY, fb * (jnp.float32(1.0) - _TINY) + _TINY)
    g = -jnp.log(-jnp.log(u))

    tot = logits + g
    tmax = jnp.max(tot, axis=1, keepdims=True)
    nxt = jnp.min(jnp.where(tot == tmax, vio, jnp.int32(_VOCAB)),
                  axis=1, keepdims=True)

    rate = jnp.sum(jnp.where(vio == nxt, inten, jnp.float32(0.0)),
                   axis=1, keepdims=True)
    accept_p = jnp.float32(1.0) - jnp.exp(-rate * _DT)

    ubits = _tf_bits(ku0, ku1, m_acc)
    u_acc = jnp.maximum(jnp.float32(0.0), _bits_to_unit(ubits))

    return jnp.where(u_acc < accept_p, nxt, cur)


def _tc_main_kernel(kcat_ref, kunif_ref, x_ref, table_ref, w1t_ref, b1_ref,
                    w2t_ref, b2_ref, out_ref):
    j = pl.program_id(0)
    cur = x_ref[0]
    vio = lax.broadcasted_iota(jnp.int32, (_B, _VOCAB), 1)
    bio = lax.broadcasted_iota(jnp.int32, (_B, _VOCAB), 0)
    m_cat = bio * jnp.int32(_VOCAB) + vio
    m_acc = lax.broadcasted_iota(jnp.int32, (_B, 1), 0)

    for i in range(_STEPS):
        bits = _tf_bits(kcat_ref[i, j, 0], kcat_ref[i, j, 1], m_cat)
        cur = _one_step(cur, bits, kunif_ref[i, j, 0], kunif_ref[i, j, 1],
                        vio, m_acc, table_ref[...], w1t_ref[...], b1_ref[0],
                        w2t_ref[...], b2_ref[0])
    out_ref[0] = cur


def _tc_consume_kernel(kunif_ref, x_ref, b0_ref, b1_ref_, b2_ref_, b3_ref,
                       b4_ref, table_ref, w1t_ref, b1_ref, w2t_ref, b2_ref,
                       out_ref):
    j = pl.program_id(0)
    cur = x_ref[0]
    vio = lax.broadcasted_iota(jnp.int32, (_B, _VOCAB), 1)
    m_acc = lax.broadcasted_iota(jnp.int32, (_B, 1), 0)
    bits_refs = (b0_ref, b1_ref_, b2_ref_, b3_ref, b4_ref)
    for i in range(_STEPS):
        cur = _one_step(cur, bits_refs[i][0], kunif_ref[i, j, 0],
                        kunif_ref[i, j, 1], vio, m_acc, table_ref[...],
                        w1t_ref[...], b1_ref[0], w2t_ref[...], b2_ref[0])
    out_ref[0] = cur


def _make_sc_bits_kernel(step):
    nc, ns = 2, 16
    nw = nc * ns
    rows = _B // nw
    tile_words = rows * _VOCAB
    pos_words = _B * _VOCAB
    keys = [(int(_KCAT[step, p, 0]), int(_KCAT[step, p, 1]))
            for p in range(_NP)]

    @functools.partial(
        pl.kernel,
        mesh=plsc.VectorSubcoreMesh(core_axis_name="c", subcore_axis_name="s"),
        out_type=jax.ShapeDtypeStruct((_NP * pos_words,), jnp.int32),
        scratch_types=[
            pltpu.VMEM((tile_words,), jnp.int32),
            pltpu.VMEM((tile_words,), jnp.int32),
            pltpu.SemaphoreType.DMA,
            pltpu.SemaphoreType.DMA,
        ],
    )
    def sc_bits(out_hbm, buf0, buf1, sem0, sem1):
        wid = lax.axis_index("s") * nc + lax.axis_index("c")
        m_base = wid * jnp.int32(tile_words)
        lane = lax.iota(jnp.int32, 16)
        bufs = (buf0, buf1)
        sems = (sem0, sem1)
        pending = {0: None, 1: None}
        for p in range(_NP):
            which = p % 2
            if pending[which] is not None:
                pending[which].wait()
            buf = bufs[which]
            k0 = jnp.int32(keys[p][0])
            k1 = jnp.int32(keys[p][1])

            def body(t2, _, buf=buf, k0=k0, k1=k1):
                off = t2 * jnp.int32(32)
                m0 = m_base + off + lane
                for q in range(2):
                    buf[pl.ds(off + jnp.int32(16 * q), 16)] = _tf_bits(
                        k0, k1, m0 + jnp.int32(16 * q))
                return 0

            lax.fori_loop(0, tile_words // 32, body, 0)
            dst = out_hbm.at[pl.ds(p * pos_words + wid * tile_words,
                                   tile_words)]
            pending[which] = pltpu.async_copy(buf, dst, sems[which])
        for which in (0, 1):
            if pending[which] is not None:
                pending[which].wait()

    return sc_bits


_sc_bits_cached = functools.cache(_make_sc_bits_kernel)

_WEIGHT_SPECS = [
    pl.BlockSpec((_VOCAB, _EMB), lambda j: (0, 0)),
    pl.BlockSpec((_EMB, _HID), lambda j: (0, 0)),
    pl.BlockSpec((1, _HID), lambda j: (0, 0)),
    pl.BlockSpec((_HID, _VOCAB), lambda j: (0, 0)),
    pl.BlockSpec((1, _VOCAB), lambda j: (0, 0)),
]


def kernel(x, emb_table, W1, b1, W2, b2):
    B, L = x.shape
    xT = x.T.reshape(L, B, 1)
    w1t = W1.T
    w2t = W2.T
    b1r = b1.reshape(1, _HID)
    b2r = b2.reshape(1, _VOCAB)
    weights = (emb_table, w1t, b1r, w2t, b2r)

    n_main = L - _NP
    a_out = pl.pallas_call(
        _tc_main_kernel,
        grid=(n_main,),
        in_specs=[
            pl.BlockSpec(memory_space=pltpu.SMEM),
            pl.BlockSpec(memory_space=pltpu.SMEM),
            pl.BlockSpec((1, B, 1), lambda j: (j, 0, 0)),
            *_WEIGHT_SPECS,
        ],
        out_specs=pl.BlockSpec((1, B, 1), lambda j: (j, 0, 0)),
        out_shape=jax.ShapeDtypeStruct((n_main, B, 1), jnp.int32),
    )(jnp.asarray(_KCAT[:, _NP:]), jnp.asarray(_KUNIF[:, _NP:]), xT[_NP:],
      *weights)

    sc_bits = [
        _sc_bits_cached(i)().reshape(_NP, B, _VOCAB) for i in range(_STEPS)
    ]

    xb = xT[:_NP] + a_out[0, 0, 0] * 0

    xb = pl.pallas_call(
        _tc_consume_kernel,
        grid=(_NP,),
        in_specs=[
            pl.BlockSpec(memory_space=pltpu.SMEM),
            pl.BlockSpec((1, B, 1), lambda j: (j, 0, 0)),
            *[pl.BlockSpec((1, B, _VOCAB), lambda j: (j, 0, 0))
              for _ in range(_STEPS)],
            *_WEIGHT_SPECS,
        ],
        out_specs=pl.BlockSpec((1, B, 1), lambda j: (j, 0, 0)),
        out_shape=jax.ShapeDtypeStruct((_NP, B, 1), jnp.int32),
    )(jnp.asarray(_KUNIF[:, :_NP]), xb, *sc_bits, *weights)

    return jnp.concatenate([xb, a_out], axis=0).reshape(L, B).T


# speedup vs baseline: 1.0187x; 1.0187x over previous
import functools

import numpy as np
import jax
import jax.numpy as jnp
from jax import lax
from jax.experimental import pallas as pl
from jax.experimental.pallas import tpu as pltpu
from jax.experimental.pallas import tpu_sc as plsc

_VOCAB = 1000
_EMB = 128
_HID = 64
_STEPS = 5
_L = 50
_B = 1024
_NP = 20
_DT = np.float32(0.05 / 5)
_TINY = np.float32(np.finfo(np.float32).tiny)

_ROTS = ((13, 15, 26, 6), (17, 29, 16, 24))


def _np_threefry_block(k0, k1, x0, x1):
    k0 = np.uint32(k0)
    k1 = np.uint32(k1)
    ks = (k0, k1, k0 ^ k1 ^ np.uint32(0x1BD11BDA))
    x0 = np.asarray(x0, np.uint32)
    x1 = np.asarray(x1, np.uint32)
    with np.errstate(over="ignore"):
        x0 = x0 + ks[0]
        x1 = x1 + ks[1]
        for i in range(5):
            for r in _ROTS[i % 2]:
                x0 = x0 + x1
                x1 = (x1 << np.uint32(r)) | (x1 >> np.uint32(32 - r))
                x1 = x1 ^ x0
            x0 = x0 + ks[(i + 1) % 3]
            x1 = x1 + ks[(i + 2) % 3] + np.uint32(i + 1)
    return x0, x1


def _np_fold_in(kp, data):
    w0, w1 = _np_threefry_block(kp[0], kp[1], np.uint32(0), np.uint32(data))
    return (w0, w1)


def _derive_keys():
    base = (np.uint32(0), np.uint32(42))
    kcat = np.zeros((_STEPS, _L, 2), np.uint32)
    kunif = np.zeros((_STEPS, _L, 2), np.uint32)
    for i in range(_STEPS):
        ki = _np_fold_in(base, i)
        for j in range(_L):
            kk = _np_fold_in(ki, j)
            w0, w1 = _np_threefry_block(
                kk[0], kk[1], np.zeros(2, np.uint32), np.arange(2, dtype=np.uint32)
            )
            kcat[i, j] = (w0[0], w1[0])
            kunif[i, j] = (w0[1], w1[1])
    return kcat.view(np.int32), kunif.view(np.int32)


_KCAT, _KUNIF = _derive_keys()


def _tf_bits(k0, k1, m):
    ks2 = k0 ^ k1 ^ jnp.int32(0x1BD11BDA)
    ks = (k0, k1, ks2)
    x0 = jnp.zeros_like(m) + k0
    x1 = m + k1
    for i in range(5):
        for r in _ROTS[i % 2]:
            x0 = x0 + x1
            x1 = lax.shift_left(x1, jnp.int32(r)) | lax.shift_right_logical(
                x1, jnp.int32(32 - r)
            )
            x1 = x1 ^ x0
        x0 = x0 + ks[(i + 1) % 3]
        x1 = x1 + ks[(i + 2) % 3] + jnp.int32(i + 1)
    return x0 ^ x1


def _bits_to_unit(bits):
    fb = lax.shift_right_logical(bits, jnp.int32(9)) | jnp.int32(0x3F800000)
    return lax.bitcast_convert_type(fb, jnp.float32) - jnp.float32(1.0)


def _one_step(cur, cat_bits, ku0, ku1, vio, m_acc, table, w1t, b1, w2t, b2):
    onehot_b = vio == cur
    oh = onehot_b.astype(jnp.float32)
    emb = jnp.dot(oh, table, preferred_element_type=jnp.float32)
    h = jnp.dot(emb, w1t, preferred_element_type=jnp.float32)
    h = jnp.maximum(h + b1, jnp.float32(0.0))
    inten = jnp.dot(h, w2t, preferred_element_type=jnp.float32)
    inten = inten + b2

    ctr = inten - jnp.max(inten, axis=1, keepdims=True)
    e = jnp.exp(ctr)
    s = jnp.sum(e, axis=1, keepdims=True)
    em = jnp.where(onehot_b, jnp.float32(0.0), e)
    s2 = jnp.maximum(jnp.sum(em, axis=1, keepdims=True) / s, jnp.float32(1e-8))
    lsum = jnp.log(s) + jnp.log(s2)
    c = jnp.float32(np.log(np.float32(1e-12)))
    logits = jnp.where(onehot_b, c, jnp.maximum(ctr - lsum, c))

    fb = _bits_to_unit(cat_bits)
    u = jnp.maximum(_TINY, fb * (jnp.float32(1.0) - _TINY) + _TINY)
    g = -jnp.log(-jnp.log(u))

    tot = logits + g
    tmax = jnp.max(tot, axis=1, keepdims=True)
    nxt = jnp.min(jnp.where(tot == tmax, vio, jnp.int32(_VOCAB)),
                  axis=1, keepdims=True)

    rate = jnp.sum(jnp.where(vio == nxt, inten, jnp.float32(0.0)),
                   axis=1, keepdims=True)
    accept_p = jnp.float32(1.0) - jnp.exp(-rate * _DT)

    ubits = _tf_bits(ku0, ku1, m_acc)
    u_acc = jnp.maximum(jnp.float32(0.0), _bits_to_unit(ubits))

    return jnp.where(u_acc < accept_p, nxt, cur)


def _tc_main_kernel(kcat_ref, kunif_ref, x_ref, table_ref, w1t_ref, b1_ref,
                    w2t_ref, b2_ref, out_ref):
    j = pl.program_id(0)
    cur = x_ref[0]
    vio = lax.broadcasted_iota(jnp.int32, (_B, _VOCAB), 1)
    bio = lax.broadcasted_iota(jnp.int32, (_B, _VOCAB), 0)
    m_cat = bio * jnp.int32(_VOCAB) + vio
    m_acc = lax.broadcasted_iota(jnp.int32, (_B, 1), 0)

    for i in range(_STEPS):
        bits = _tf_bits(kcat_ref[i, j, 0], kcat_ref[i, j, 1], m_cat)
        cur = _one_step(cur, bits, kunif_ref[i, j, 0], kunif_ref[i, j, 1],
                        vio, m_acc, table_ref[...], w1t_ref[...], b1_ref[0],
                        w2t_ref[...], b2_ref[0])
    out_ref[0] = cur


def _tc_consume_kernel(kunif_ref, x_ref, b0_ref, b1_ref_, b2_ref_, b3_ref,
                       b4_ref, table_ref, w1t_ref, b1_ref, w2t_ref, b2_ref,
                       out_ref):
    j = pl.program_id(0)
    cur = x_ref[0]
    vio = lax.broadcasted_iota(jnp.int32, (_B, _VOCAB), 1)
    m_acc = lax.broadcasted_iota(jnp.int32, (_B, 1), 0)
    bits_refs = (b0_ref, b1_ref_, b2_ref_, b3_ref, b4_ref)
    for i in range(_STEPS):
        cur = _one_step(cur, bits_refs[i][0], kunif_ref[i, j, 0],
                        kunif_ref[i, j, 1], vio, m_acc, table_ref[...],
                        w1t_ref[...], b1_ref[0], w2t_ref[...], b2_ref[0])
    out_ref[0] = cur


def _make_sc_bits_kernel(step):
    nc, ns = 2, 16
    nw = nc * ns
    rows = _B // nw
    tile_words = rows * _VOCAB
    pos_words = _B * _VOCAB
    keys = [(int(_KCAT[step, p, 0]), int(_KCAT[step, p, 1]))
            for p in range(_NP)]

    @functools.partial(
        pl.kernel,
        mesh=plsc.VectorSubcoreMesh(core_axis_name="c", subcore_axis_name="s"),
        out_type=jax.ShapeDtypeStruct((_NP * pos_words,), jnp.int32),
        scratch_types=[
            pltpu.VMEM((tile_words,), jnp.int32),
            pltpu.VMEM((tile_words,), jnp.int32),
            pltpu.SemaphoreType.DMA,
            pltpu.SemaphoreType.DMA,
        ],
    )
    def sc_bits(out_hbm, buf0, buf1, sem0, sem1):
        wid = lax.axis_index("s") * nc + lax.axis_index("c")
        m_base = wid * jnp.int32(tile_words)
        lane = lax.iota(jnp.int32, 16)
        bufs = (buf0, buf1)
        sems = (sem0, sem1)
        pending = {0: None, 1: None}
        for p in range(_NP):
            which = p % 2
            if pending[which] is not None:
                pending[which].wait()
            buf = bufs[which]
            k0 = jnp.int32(keys[p][0])
            k1 = jnp.int32(keys[p][1])

            def body(t2, _, buf=buf, k0=k0, k1=k1):
                off = t2 * jnp.int32(32)
                m0 = m_base + off + lane
                for q in range(2):
                    buf[pl.ds(off + jnp.int32(16 * q), 16)] = _tf_bits(
                        k0, k1, m0 + jnp.int32(16 * q))
                return 0

            lax.fori_loop(0, tile_words // 32, body, 0)
            dst = out_hbm.at[pl.ds(p * pos_words + wid * tile_words,
                                   tile_words)]
            pending[which] = pltpu.async_copy(buf, dst, sems[which])
        for which in (0, 1):
            if pending[which] is not None:
                pending[which].wait()

    return sc_bits


_sc_bits_cached = functools.cache(_make_sc_bits_kernel)

_WEIGHT_SPECS = [
    pl.BlockSpec((_VOCAB, _EMB), lambda j: (0, 0)),
    pl.BlockSpec((_EMB, _HID), lambda j: (0, 0)),
    pl.BlockSpec((1, _HID), lambda j: (0, 0)),
    pl.BlockSpec((_HID, _VOCAB), lambda j: (0, 0)),
    pl.BlockSpec((1, _VOCAB), lambda j: (0, 0)),
]


def kernel(x, emb_table, W1, b1, W2, b2):
    B, L = x.shape
    xT = x.T.reshape(L, B, 1)
    w1t = W1.T
    w2t = W2.T
    b1r = b1.reshape(1, _HID)
    b2r = b2.reshape(1, _VOCAB)
    weights = (emb_table, w1t, b1r, w2t, b2r)

    n_main = L - _NP
    a_out = pl.pallas_call(
        _tc_main_kernel,
        grid=(n_main,),
        in_specs=[
            pl.BlockSpec(memory_space=pltpu.SMEM),
            pl.BlockSpec(memory_space=pltpu.SMEM),
            pl.BlockSpec((1, B, 1), lambda j: (j, 0, 0)),
            *_WEIGHT_SPECS,
        ],
        out_specs=pl.BlockSpec((1, B, 1), lambda j: (j, 0, 0)),
        out_shape=jax.ShapeDtypeStruct((n_main, B, 1), jnp.int32),
    )(jnp.asarray(_KCAT[:, _NP:]), jnp.asarray(_KUNIF[:, _NP:]), xT[_NP:],
      *weights)

    sc_bits = [
        _sc_bits_cached(i)().reshape(_NP, B, _VOCAB) for i in range(_STEPS)
    ]

    xb = xT[:_NP] + a_out[0, 0, 0] * 0

    xb = pl.pallas_call(
        _tc_consume_kernel,
        grid=(_NP,),
        in_specs=[
            pl.BlockSpec(memory_space=pltpu.SMEM),
            pl.BlockSpec((1, B, 1), lambda j: (j, 0, 0)),
            *[pl.BlockSpec((1, B, _VOCAB), lambda j: (j, 0, 0))
              for _ in range(_STEPS)],
            *_WEIGHT_SPECS,
        ],
        out_specs=pl.BlockSpec((1, B, 1), lambda j: (j, 0, 0)),
        out_shape=jax.ShapeDtypeStruct((_NP, B, 1), jnp.int32),
    )(jnp.asarray(_KUNIF[:, :_NP]), xb, *sc_bits, *weights)

    return jnp.concatenate([xb, a_out], axis=0).reshape(L, B).T


# speedup vs baseline: 1.0193x; 1.0006x over previous
import functools

import numpy as np
import jax
import jax.numpy as jnp
from jax import lax
from jax.experimental import pallas as pl
from jax.experimental.pallas import tpu as pltpu
from jax.experimental.pallas import tpu_sc as plsc

_VOCAB = 1000
_EMB = 128
_HID = 64
_STEPS = 5
_L = 50
_B = 1024
_NP = 20
_DT = np.float32(0.05 / 5)
_TINY = np.float32(np.finfo(np.float32).tiny)

_ROTS = ((13, 15, 26, 6), (17, 29, 16, 24))


def _np_threefry_block(k0, k1, x0, x1):
    k0 = np.uint32(k0)
    k1 = np.uint32(k1)
    ks = (k0, k1, k0 ^ k1 ^ np.uint32(0x1BD11BDA))
    x0 = np.asarray(x0, np.uint32)
    x1 = np.asarray(x1, np.uint32)
    with np.errstate(over="ignore"):
        x0 = x0 + ks[0]
        x1 = x1 + ks[1]
        for i in range(5):
            for r in _ROTS[i % 2]:
                x0 = x0 + x1
                x1 = (x1 << np.uint32(r)) | (x1 >> np.uint32(32 - r))
                x1 = x1 ^ x0
            x0 = x0 + ks[(i + 1) % 3]
            x1 = x1 + ks[(i + 2) % 3] + np.uint32(i + 1)
    return x0, x1


def _np_fold_in(kp, data):
    w0, w1 = _np_threefry_block(kp[0], kp[1], np.uint32(0), np.uint32(data))
    return (w0, w1)


def _derive_keys():
    base = (np.uint32(0), np.uint32(42))
    kcat = np.zeros((_STEPS, _L, 2), np.uint32)
    kunif = np.zeros((_STEPS, _L, 2), np.uint32)
    for i in range(_STEPS):
        ki = _np_fold_in(base, i)
        for j in range(_L):
            kk = _np_fold_in(ki, j)
            w0, w1 = _np_threefry_block(
                kk[0], kk[1], np.zeros(2, np.uint32), np.arange(2, dtype=np.uint32)
            )
            kcat[i, j] = (w0[0], w1[0])
            kunif[i, j] = (w0[1], w1[1])
    return kcat.view(np.int32), kunif.view(np.int32)


_KCAT, _KUNIF = _derive_keys()


def _tf_bits(k0, k1, m):
    ks2 = k0 ^ k1 ^ jnp.int32(0x1BD11BDA)
    ks = (k0, k1, ks2)
    x0 = jnp.zeros_like(m) + k0
    x1 = m + k1
    for i in range(5):
        for r in _ROTS[i % 2]:
            x0 = x0 + x1
            x1 = lax.shift_left(x1, jnp.int32(r)) | lax.shift_right_logical(
                x1, jnp.int32(32 - r)
            )
            x1 = x1 ^ x0
        x0 = x0 + ks[(i + 1) % 3]
        x1 = x1 + ks[(i + 2) % 3] + jnp.int32(i + 1)
    return x0 ^ x1


def _bits_to_unit(bits):
    fb = lax.shift_right_logical(bits, jnp.int32(9)) | jnp.int32(0x3F800000)
    return lax.bitcast_convert_type(fb, jnp.float32) - jnp.float32(1.0)


def _one_step(cur, cat_bits, ku0, ku1, vio, m_acc, table, w1t, b1, w2t, b2):
    onehot_b = vio == cur
    oh = onehot_b.astype(jnp.float32)
    emb = jnp.dot(oh, table, preferred_element_type=jnp.float32)
    h = jnp.dot(emb, w1t, preferred_element_type=jnp.float32)
    h = jnp.maximum(h + b1, jnp.float32(0.0))
    inten = jnp.dot(h, w2t, preferred_element_type=jnp.float32)
    inten = inten + b2

    ctr = inten - jnp.max(inten, axis=1, keepdims=True)
    e = jnp.exp(ctr)
    s = jnp.sum(e, axis=1, keepdims=True)
    em = jnp.where(onehot_b, jnp.float32(0.0), e)
    s2 = jnp.maximum(jnp.sum(em, axis=1, keepdims=True) / s, jnp.float32(1e-8))
    lsum = jnp.log(s) + jnp.log(s2)
    c = jnp.float32(np.log(np.float32(1e-12)))
    logits = jnp.where(onehot_b, c, jnp.maximum(ctr - lsum, c))

    fb = _bits_to_unit(cat_bits)
    u = jnp.maximum(_TINY, fb * (jnp.float32(1.0) - _TINY) + _TINY)
    g = -jnp.log(-jnp.log(u))

    tot = logits + g
    tmax = jnp.max(tot, axis=1, keepdims=True)
    nxt = jnp.min(jnp.where(tot == tmax, vio, jnp.int32(_VOCAB)),
                  axis=1, keepdims=True)

    rate = jnp.sum(jnp.where(vio == nxt, inten, jnp.float32(0.0)),
                   axis=1, keepdims=True)
    accept_p = jnp.float32(1.0) - jnp.exp(-rate * _DT)

    ubits = _tf_bits(ku0, ku1, m_acc)
    u_acc = jnp.maximum(jnp.float32(0.0), _bits_to_unit(ubits))

    return jnp.where(u_acc < accept_p, nxt, cur)


def _tc_main_kernel(kcat_ref, kunif_ref, x_ref, table_ref, w1t_ref, b1_ref,
                    w2t_ref, b2_ref, out_ref):
    j = pl.program_id(0)
    cur = x_ref[0]
    vio = lax.broadcasted_iota(jnp.int32, (_B, _VOCAB), 1)
    bio = lax.broadcasted_iota(jnp.int32, (_B, _VOCAB), 0)
    m_cat = bio * jnp.int32(_VOCAB) + vio
    m_acc = lax.broadcasted_iota(jnp.int32, (_B, 1), 0)

    for i in range(_STEPS):
        bits = _tf_bits(kcat_ref[i, j, 0], kcat_ref[i, j, 1], m_cat)
        cur = _one_step(cur, bits, kunif_ref[i, j, 0], kunif_ref[i, j, 1],
                        vio, m_acc, table_ref[...], w1t_ref[...], b1_ref[0],
                        w2t_ref[...], b2_ref[0])
    out_ref[0] = cur


def _tc_consume_kernel(kunif_ref, x_ref, b0_ref, b1_ref_, b2_ref_, b3_ref,
                       b4_ref, table_ref, w1t_ref, b1_ref, w2t_ref, b2_ref,
                       out_ref):
    j = pl.program_id(0)
    cur = x_ref[0]
    vio = lax.broadcasted_iota(jnp.int32, (_B, _VOCAB), 1)
    m_acc = lax.broadcasted_iota(jnp.int32, (_B, 1), 0)
    bits_refs = (b0_ref, b1_ref_, b2_ref_, b3_ref, b4_ref)
    for i in range(_STEPS):
        cur = _one_step(cur, bits_refs[i][0], kunif_ref[i, j, 0],
                        kunif_ref[i, j, 1], vio, m_acc, table_ref[...],
                        w1t_ref[...], b1_ref[0], w2t_ref[...], b2_ref[0])
    out_ref[0] = cur


def _make_sc_bits_kernel(step):
    nc, ns = 2, 16
    nw = nc * ns
    rows = _B // nw
    tile_words = rows * _VOCAB
    pos_words = _B * _VOCAB
    keys = [(int(_KCAT[step, p, 0]), int(_KCAT[step, p, 1]))
            for p in range(_NP)]

    @functools.partial(
        pl.kernel,
        mesh=plsc.VectorSubcoreMesh(core_axis_name="c", subcore_axis_name="s"),
        out_type=jax.ShapeDtypeStruct((_NP * pos_words,), jnp.int32),
        scratch_types=[
            pltpu.VMEM((tile_words + 16,), jnp.int32),
            pltpu.VMEM((tile_words + 16,), jnp.int32),
            pltpu.SemaphoreType.DMA,
            pltpu.SemaphoreType.DMA,
        ],
    )
    def sc_bits(out_hbm, buf0, buf1, sem0, sem1):
        wid = lax.axis_index("s") * nc + lax.axis_index("c")
        m_base = wid * jnp.int32(tile_words)
        lane = lax.iota(jnp.int32, 16)
        bufs = (buf0, buf1)
        sems = (sem0, sem1)
        pending = {0: None, 1: None}
        for p in range(_NP):
            which = p % 2
            if pending[which] is not None:
                pending[which].wait()
            buf = bufs[which]
            k0 = jnp.int32(keys[p][0])
            k1 = jnp.int32(keys[p][1])

            def body(t2, _, buf=buf, k0=k0, k1=k1):
                off = t2 * jnp.int32(48)
                m0 = m_base + off + lane
                for q in range(3):
                    buf[pl.ds(off + jnp.int32(16 * q), 16)] = _tf_bits(
                        k0, k1, m0 + jnp.int32(16 * q))
                return 0

            lax.fori_loop(0, (tile_words + 47) // 48, body, 0)
            dst = out_hbm.at[pl.ds(p * pos_words + wid * tile_words,
                                   tile_words)]
            pending[which] = pltpu.async_copy(
                buf.at[pl.ds(0, tile_words)], dst, sems[which])
        for which in (0, 1):
            if pending[which] is not None:
                pending[which].wait()

    return sc_bits


_sc_bits_cached = functools.cache(_make_sc_bits_kernel)

_WEIGHT_SPECS = [
    pl.BlockSpec((_VOCAB, _EMB), lambda j: (0, 0)),
    pl.BlockSpec((_EMB, _HID), lambda j: (0, 0)),
    pl.BlockSpec((1, _HID), lambda j: (0, 0)),
    pl.BlockSpec((_HID, _VOCAB), lambda j: (0, 0)),
    pl.BlockSpec((1, _VOCAB), lambda j: (0, 0)),
]


def kernel(x, emb_table, W1, b1, W2, b2):
    B, L = x.shape
    xT = x.T.reshape(L, B, 1)
    w1t = W1.T
    w2t = W2.T
    b1r = b1.reshape(1, _HID)
    b2r = b2.reshape(1, _VOCAB)
    weights = (emb_table, w1t, b1r, w2t, b2r)

    n_main = L - _NP
    a_out = pl.pallas_call(
        _tc_main_kernel,
        grid=(n_main,),
        in_specs=[
            pl.BlockSpec(memory_space=pltpu.SMEM),
            pl.BlockSpec(memory_space=pltpu.SMEM),
            pl.BlockSpec((1, B, 1), lambda j: (j, 0, 0)),
            *_WEIGHT_SPECS,
        ],
        out_specs=pl.BlockSpec((1, B, 1), lambda j: (j, 0, 0)),
        out_shape=jax.ShapeDtypeStruct((n_main, B, 1), jnp.int32),
    )(jnp.asarray(_KCAT[:, _NP:]), jnp.asarray(_KUNIF[:, _NP:]), xT[_NP:],
      *weights)

    sc_bits = [
        _sc_bits_cached(i)().reshape(_NP, B, _VOCAB) for i in range(_STEPS)
    ]

    xb = xT[:_NP] + a_out[0, 0, 0] * 0

    xb = pl.pallas_call(
        _tc_consume_kernel,
        grid=(_NP,),
        in_specs=[
            pl.BlockSpec(memory_space=pltpu.SMEM),
            pl.BlockSpec((1, B, 1), lambda j: (j, 0, 0)),
            *[pl.BlockSpec((1, B, _VOCAB), lambda j: (j, 0, 0))
              for _ in range(_STEPS)],
            *_WEIGHT_SPECS,
        ],
        out_specs=pl.BlockSpec((1, B, 1), lambda j: (j, 0, 0)),
        out_shape=jax.ShapeDtypeStruct((_NP, B, 1), jnp.int32),
    )(jnp.asarray(_KUNIF[:, :_NP]), xb, *sc_bits, *weights)

    return jnp.concatenate([xb, a_out], axis=0).reshape(L, B).T


# speedup vs baseline: 1.0532x; 1.0332x over previous
import functools

import numpy as np
import jax
import jax.numpy as jnp
from jax import lax
from jax.experimental import pallas as pl
from jax.experimental.pallas import tpu as pltpu
from jax.experimental.pallas import tpu_sc as plsc

_VOCAB = 1000
_EMB = 128
_HID = 64
_STEPS = 5
_L = 50
_B = 1024
_NP = 20
_DT = np.float32(0.05 / 5)
_TINY = np.float32(np.finfo(np.float32).tiny)

_ROTS = ((13, 15, 26, 6), (17, 29, 16, 24))


def _np_threefry_block(k0, k1, x0, x1):
    k0 = np.uint32(k0)
    k1 = np.uint32(k1)
    ks = (k0, k1, k0 ^ k1 ^ np.uint32(0x1BD11BDA))
    x0 = np.asarray(x0, np.uint32)
    x1 = np.asarray(x1, np.uint32)
    with np.errstate(over="ignore"):
        x0 = x0 + ks[0]
        x1 = x1 + ks[1]
        for i in range(5):
            for r in _ROTS[i % 2]:
                x0 = x0 + x1
                x1 = (x1 << np.uint32(r)) | (x1 >> np.uint32(32 - r))
                x1 = x1 ^ x0
            x0 = x0 + ks[(i + 1) % 3]
            x1 = x1 + ks[(i + 2) % 3] + np.uint32(i + 1)
    return x0, x1


def _np_fold_in(kp, data):
    w0, w1 = _np_threefry_block(kp[0], kp[1], np.uint32(0), np.uint32(data))
    return (w0, w1)


def _derive_keys():
    base = (np.uint32(0), np.uint32(42))
    kcat = np.zeros((_STEPS, _L, 2), np.uint32)
    kunif = np.zeros((_STEPS, _L, 2), np.uint32)
    for i in range(_STEPS):
        ki = _np_fold_in(base, i)
        for j in range(_L):
            kk = _np_fold_in(ki, j)
            w0, w1 = _np_threefry_block(
                kk[0], kk[1], np.zeros(2, np.uint32), np.arange(2, dtype=np.uint32)
            )
            kcat[i, j] = (w0[0], w1[0])
            kunif[i, j] = (w0[1], w1[1])
    return kcat.view(np.int32), kunif.view(np.int32)


_KCAT, _KUNIF = _derive_keys()


def _tf_bits(k0, k1, m):
    ks2 = k0 ^ k1 ^ jnp.int32(0x1BD11BDA)
    ks = (k0, k1, ks2)
    x0 = jnp.zeros_like(m) + k0
    x1 = m + k1
    for i in range(5):
        for r in _ROTS[i % 2]:
            x0 = x0 + x1
            x1 = lax.shift_left(x1, jnp.int32(r)) | lax.shift_right_logical(
                x1, jnp.int32(32 - r)
            )
            x1 = x1 ^ x0
        x0 = x0 + ks[(i + 1) % 3]
        x1 = x1 + ks[(i + 2) % 3] + jnp.int32(i + 1)
    return x0 ^ x1


def _bits_to_unit(bits):
    fb = lax.shift_right_logical(bits, jnp.int32(9)) | jnp.int32(0x3F800000)
    return lax.bitcast_convert_type(fb, jnp.float32) - jnp.float32(1.0)


def _one_step(cur, cat_bits, ku0, ku1, vio, m_acc, table, w1t, b1, w2t, b2):
    onehot_b = vio == cur
    oh = onehot_b.astype(jnp.float32)
    emb = jnp.dot(oh, table, preferred_element_type=jnp.float32)
    h = jnp.dot(emb, w1t, preferred_element_type=jnp.float32)
    h = jnp.maximum(h + b1, jnp.float32(0.0))
    inten = jnp.dot(h, w2t, preferred_element_type=jnp.float32)
    inten = inten + b2

    ctr = inten - jnp.max(inten, axis=1, keepdims=True)
    e = jnp.exp(ctr)
    s = jnp.sum(e, axis=1, keepdims=True)
    em = jnp.where(onehot_b, jnp.float32(0.0), e)
    s2 = jnp.maximum(jnp.sum(em, axis=1, keepdims=True) / s, jnp.float32(1e-8))
    lsum = jnp.log(s) + jnp.log(s2)
    c = jnp.float32(np.log(np.float32(1e-12)))
    logits = jnp.where(onehot_b, c, jnp.maximum(ctr - lsum, c))

    fb = _bits_to_unit(cat_bits)
    u = jnp.maximum(_TINY, fb * (jnp.float32(1.0) - _TINY) + _TINY)
    g = -jnp.log(-jnp.log(u))

    tot = logits + g
    tmax = jnp.max(tot, axis=1, keepdims=True)
    nxt = jnp.min(jnp.where(tot == tmax, vio, jnp.int32(_VOCAB)),
                  axis=1, keepdims=True)

    rate = jnp.sum(jnp.where(vio == nxt, inten, jnp.float32(0.0)),
                   axis=1, keepdims=True)
    accept_p = jnp.float32(1.0) - jnp.exp(-rate * _DT)

    ubits = _tf_bits(ku0, ku1, m_acc)
    u_acc = jnp.maximum(jnp.float32(0.0), _bits_to_unit(ubits))

    return jnp.where(u_acc < accept_p, nxt, cur)


def _tc_main_kernel(kcat_ref, kunif_ref, x_ref, table_ref, w1t_ref, b1_ref,
                    w2t_ref, b2_ref, out_ref):
    j = pl.program_id(0)
    cur = x_ref[0]
    vio = lax.broadcasted_iota(jnp.int32, (_B, _VOCAB), 1)
    bio = lax.broadcasted_iota(jnp.int32, (_B, _VOCAB), 0)
    m_cat = bio * jnp.int32(_VOCAB) + vio
    m_acc = lax.broadcasted_iota(jnp.int32, (_B, 1), 0)

    for i in range(_STEPS):
        bits = _tf_bits(kcat_ref[i, j, 0], kcat_ref[i, j, 1], m_cat)
        cur = _one_step(cur, bits, kunif_ref[i, j, 0], kunif_ref[i, j, 1],
                        vio, m_acc, table_ref[...], w1t_ref[...], b1_ref[0],
                        w2t_ref[...], b2_ref[0])
    out_ref[0] = cur


def _tc_consume_kernel(kunif_ref, x_ref, bits_ref, table_ref, w1t_ref, b1_ref,
                       w2t_ref, b2_ref, out_ref):
    j = pl.program_id(0)
    cur = x_ref[0]
    vio = lax.broadcasted_iota(jnp.int32, (_B, _VOCAB), 1)
    m_acc = lax.broadcasted_iota(jnp.int32, (_B, 1), 0)
    out_ref[0] = _one_step(cur, bits_ref[0], kunif_ref[j, 0], kunif_ref[j, 1],
                           vio, m_acc, table_ref[...], w1t_ref[...], b1_ref[0],
                           w2t_ref[...], b2_ref[0])


def _make_sc_bits_kernel(step):
    nc, ns = 2, 16
    nw = nc * ns
    rows = _B // nw
    tile_words = rows * _VOCAB
    pos_words = _B * _VOCAB
    keys = [(int(_KCAT[step, p, 0]), int(_KCAT[step, p, 1]))
            for p in range(_NP)]

    @functools.partial(
        pl.kernel,
        mesh=plsc.VectorSubcoreMesh(core_axis_name="c", subcore_axis_name="s"),
        out_type=jax.ShapeDtypeStruct((_NP * pos_words,), jnp.int32),
        scratch_types=[
            pltpu.VMEM((tile_words + 16,), jnp.int32),
            pltpu.VMEM((tile_words + 16,), jnp.int32),
            pltpu.SemaphoreType.DMA,
            pltpu.SemaphoreType.DMA,
        ],
    )
    def sc_bits(out_hbm, buf0, buf1, sem0, sem1):
        wid = lax.axis_index("s") * nc + lax.axis_index("c")
        m_base = wid * jnp.int32(tile_words)
        lane = lax.iota(jnp.int32, 16)
        bufs = (buf0, buf1)
        sems = (sem0, sem1)
        pending = {0: None, 1: None}
        for p in range(_NP):
            which = p % 2
            if pending[which] is not None:
                pending[which].wait()
            buf = bufs[which]
            k0 = jnp.int32(keys[p][0])
            k1 = jnp.int32(keys[p][1])

            def body(t2, _, buf=buf, k0=k0, k1=k1):
                off = t2 * jnp.int32(48)
                m0 = m_base + off + lane
                for q in range(3):
                    buf[pl.ds(off + jnp.int32(16 * q), 16)] = _tf_bits(
                        k0, k1, m0 + jnp.int32(16 * q))
                return 0

            lax.fori_loop(0, (tile_words + 47) // 48, body, 0)
            dst = out_hbm.at[pl.ds(p * pos_words + wid * tile_words,
                                   tile_words)]
            pending[which] = pltpu.async_copy(
                buf.at[pl.ds(0, tile_words)], dst, sems[which])
        for which in (0, 1):
            if pending[which] is not None:
                pending[which].wait()

    return sc_bits


_sc_bits_cached = functools.cache(_make_sc_bits_kernel)

_WEIGHT_SPECS = [
    pl.BlockSpec((_VOCAB, _EMB), lambda j: (0, 0)),
    pl.BlockSpec((_EMB, _HID), lambda j: (0, 0)),
    pl.BlockSpec((1, _HID), lambda j: (0, 0)),
    pl.BlockSpec((_HID, _VOCAB), lambda j: (0, 0)),
    pl.BlockSpec((1, _VOCAB), lambda j: (0, 0)),
]


def kernel(x, emb_table, W1, b1, W2, b2):
    B, L = x.shape
    xT = x.T.reshape(L, B, 1)
    w1t = W1.T
    w2t = W2.T
    b1r = b1.reshape(1, _HID)
    b2r = b2.reshape(1, _VOCAB)
    weights = (emb_table, w1t, b1r, w2t, b2r)

    n_main = L - _NP
    a_out = pl.pallas_call(
        _tc_main_kernel,
        grid=(n_main,),
        in_specs=[
            pl.BlockSpec(memory_space=pltpu.SMEM),
            pl.BlockSpec(memory_space=pltpu.SMEM),
            pl.BlockSpec((1, B, 1), lambda j: (j, 0, 0)),
            *_WEIGHT_SPECS,
        ],
        out_specs=pl.BlockSpec((1, B, 1), lambda j: (j, 0, 0)),
        out_shape=jax.ShapeDtypeStruct((n_main, B, 1), jnp.int32),
    )(jnp.asarray(_KCAT[:, _NP:]), jnp.asarray(_KUNIF[:, _NP:]), xT[_NP:],
      *weights)

    sc_bits = [
        _sc_bits_cached(i)().reshape(_NP, B, _VOCAB) for i in range(_STEPS)
    ]

    xb = xT[:_NP] + a_out[0, 0, 0] * 0

    for i in range(_STEPS):
        xb = pl.pallas_call(
            _tc_consume_kernel,
            grid=(_NP,),
            in_specs=[
                pl.BlockSpec(memory_space=pltpu.SMEM),
                pl.BlockSpec((1, B, 1), lambda j: (j, 0, 0)),
                pl.BlockSpec((1, B, _VOCAB), lambda j: (j, 0, 0)),
                *_WEIGHT_SPECS,
            ],
            out_specs=pl.BlockSpec((1, B, 1), lambda j: (j, 0, 0)),
            out_shape=jax.ShapeDtypeStruct((_NP, B, 1), jnp.int32),
        )(jnp.asarray(_KUNIF[i, :_NP]), xb, sc_bits[i], *weights)

    return jnp.concatenate([xb, a_out], axis=0).reshape(L, B).T


# speedup vs baseline: 1.0781x; 1.0236x over previous
import functools

import numpy as np
import jax
import jax.numpy as jnp
from jax import lax
from jax.experimental import pallas as pl
from jax.experimental.pallas import tpu as pltpu
from jax.experimental.pallas import tpu_sc as plsc

_VOCAB = 1000
_EMB = 128
_HID = 64
_STEPS = 5
_L = 50
_B = 1024
_NP = 22
_DT = np.float32(0.05 / 5)
_TINY = np.float32(np.finfo(np.float32).tiny)

_ROTS = ((13, 15, 26, 6), (17, 29, 16, 24))


def _np_threefry_block(k0, k1, x0, x1):
    k0 = np.uint32(k0)
    k1 = np.uint32(k1)
    ks = (k0, k1, k0 ^ k1 ^ np.uint32(0x1BD11BDA))
    x0 = np.asarray(x0, np.uint32)
    x1 = np.asarray(x1, np.uint32)
    with np.errstate(over="ignore"):
        x0 = x0 + ks[0]
        x1 = x1 + ks[1]
        for i in range(5):
            for r in _ROTS[i % 2]:
                x0 = x0 + x1
                x1 = (x1 << np.uint32(r)) | (x1 >> np.uint32(32 - r))
                x1 = x1 ^ x0
            x0 = x0 + ks[(i + 1) % 3]
            x1 = x1 + ks[(i + 2) % 3] + np.uint32(i + 1)
    return x0, x1


def _np_fold_in(kp, data):
    w0, w1 = _np_threefry_block(kp[0], kp[1], np.uint32(0), np.uint32(data))
    return (w0, w1)


def _derive_keys():
    base = (np.uint32(0), np.uint32(42))
    kcat = np.zeros((_STEPS, _L, 2), np.uint32)
    kunif = np.zeros((_STEPS, _L, 2), np.uint32)
    for i in range(_STEPS):
        ki = _np_fold_in(base, i)
        for j in range(_L):
            kk = _np_fold_in(ki, j)
            w0, w1 = _np_threefry_block(
                kk[0], kk[1], np.zeros(2, np.uint32), np.arange(2, dtype=np.uint32)
            )
            kcat[i, j] = (w0[0], w1[0])
            kunif[i, j] = (w0[1], w1[1])
    return kcat.view(np.int32), kunif.view(np.int32)


_KCAT, _KUNIF = _derive_keys()


def _tf_bits(k0, k1, m):
    ks2 = k0 ^ k1 ^ jnp.int32(0x1BD11BDA)
    ks = (k0, k1, ks2)
    x0 = jnp.zeros_like(m) + k0
    x1 = m + k1
    for i in range(5):
        for r in _ROTS[i % 2]:
            x0 = x0 + x1
            x1 = lax.shift_left(x1, jnp.int32(r)) | lax.shift_right_logical(
                x1, jnp.int32(32 - r)
            )
            x1 = x1 ^ x0
        x0 = x0 + ks[(i + 1) % 3]
        x1 = x1 + ks[(i + 2) % 3] + jnp.int32(i + 1)
    return x0 ^ x1


def _bits_to_unit(bits):
    fb = lax.shift_right_logical(bits, jnp.int32(9)) | jnp.int32(0x3F800000)
    return lax.bitcast_convert_type(fb, jnp.float32) - jnp.float32(1.0)


def _one_step(cur, cat_bits, ku0, ku1, vio, m_acc, table, w1t, b1, w2t, b2):
    onehot_b = vio == cur
    oh = onehot_b.astype(jnp.float32)
    emb = jnp.dot(oh, table, preferred_element_type=jnp.float32)
    h = jnp.dot(emb, w1t, preferred_element_type=jnp.float32)
    h = jnp.maximum(h + b1, jnp.float32(0.0))
    inten = jnp.dot(h, w2t, preferred_element_type=jnp.float32)
    inten = inten + b2

    ctr = inten - jnp.max(inten, axis=1, keepdims=True)
    e = jnp.exp(ctr)
    s = jnp.sum(e, axis=1, keepdims=True)
    em = jnp.where(onehot_b, jnp.float32(0.0), e)
    s2 = jnp.maximum(jnp.sum(em, axis=1, keepdims=True) / s, jnp.float32(1e-8))
    lsum = jnp.log(s) + jnp.log(s2)
    c = jnp.float32(np.log(np.float32(1e-12)))
    logits = jnp.where(onehot_b, c, jnp.maximum(ctr - lsum, c))

    fb = _bits_to_unit(cat_bits)
    u = jnp.maximum(_TINY, fb * (jnp.float32(1.0) - _TINY) + _TINY)
    g = -jnp.log(-jnp.log(u))

    tot = logits + g
    tmax = jnp.max(tot, axis=1, keepdims=True)
    nxt = jnp.min(jnp.where(tot == tmax, vio, jnp.int32(_VOCAB)),
                  axis=1, keepdims=True)

    rate = jnp.sum(jnp.where(vio == nxt, inten, jnp.float32(0.0)),
                   axis=1, keepdims=True)
    accept_p = jnp.float32(1.0) - jnp.exp(-rate * _DT)

    ubits = _tf_bits(ku0, ku1, m_acc)
    u_acc = jnp.maximum(jnp.float32(0.0), _bits_to_unit(ubits))

    return jnp.where(u_acc < accept_p, nxt, cur)


def _tc_main_kernel(kcat_ref, kunif_ref, x_ref, table_ref, w1t_ref, b1_ref,
                    w2t_ref, b2_ref, out_ref):
    j = pl.program_id(0)
    cur = x_ref[0]
    vio = lax.broadcasted_iota(jnp.int32, (_B, _VOCAB), 1)
    bio = lax.broadcasted_iota(jnp.int32, (_B, _VOCAB), 0)
    m_cat = bio * jnp.int32(_VOCAB) + vio
    m_acc = lax.broadcasted_iota(jnp.int32, (_B, 1), 0)

    for i in range(_STEPS):
        bits = _tf_bits(kcat_ref[i, j, 0], kcat_ref[i, j, 1], m_cat)
        cur = _one_step(cur, bits, kunif_ref[i, j, 0], kunif_ref[i, j, 1],
                        vio, m_acc, table_ref[...], w1t_ref[...], b1_ref[0],
                        w2t_ref[...], b2_ref[0])
    out_ref[0] = cur


def _tc_consume_kernel(kunif_ref, x_ref, bits_ref, table_ref, w1t_ref, b1_ref,
                       w2t_ref, b2_ref, out_ref):
    j = pl.program_id(0)
    cur = x_ref[0]
    vio = lax.broadcasted_iota(jnp.int32, (_B, _VOCAB), 1)
    m_acc = lax.broadcasted_iota(jnp.int32, (_B, 1), 0)
    out_ref[0] = _one_step(cur, bits_ref[0], kunif_ref[j, 0], kunif_ref[j, 1],
                           vio, m_acc, table_ref[...], w1t_ref[...], b1_ref[0],
                           w2t_ref[...], b2_ref[0])


def _make_sc_bits_kernel(step):
    nc, ns = 2, 16
    nw = nc * ns
    rows = _B // nw
    tile_words = rows * _VOCAB
    pos_words = _B * _VOCAB
    keys = [(int(_KCAT[step, p, 0]), int(_KCAT[step, p, 1]))
            for p in range(_NP)]

    @functools.partial(
        pl.kernel,
        mesh=plsc.VectorSubcoreMesh(core_axis_name="c", subcore_axis_name="s"),
        out_type=jax.ShapeDtypeStruct((_NP * pos_words,), jnp.int32),
        scratch_types=[
            pltpu.VMEM((tile_words + 16,), jnp.int32),
            pltpu.VMEM((tile_words + 16,), jnp.int32),
            pltpu.SemaphoreType.DMA,
            pltpu.SemaphoreType.DMA,
        ],
    )
    def sc_bits(out_hbm, buf0, buf1, sem0, sem1):
        wid = lax.axis_index("s") * nc + lax.axis_index("c")
        m_base = wid * jnp.int32(tile_words)
        lane = lax.iota(jnp.int32, 16)
        bufs = (buf0, buf1)
        sems = (sem0, sem1)
        pending = {0: None, 1: None}
        for p in range(_NP):
            which = p % 2
            if pending[which] is not None:
                pending[which].wait()
            buf = bufs[which]
            k0 = jnp.int32(keys[p][0])
            k1 = jnp.int32(keys[p][1])

            def body(t2, _, buf=buf, k0=k0, k1=k1):
                off = t2 * jnp.int32(32)
                m0 = m_base + off + lane
                for q in range(2):
                    buf[pl.ds(off + jnp.int32(16 * q), 16)] = _tf_bits(
                        k0, k1, m0 + jnp.int32(16 * q))
                return 0

            lax.fori_loop(0, tile_words // 32, body, 0)
            dst = out_hbm.at[pl.ds(p * pos_words + wid * tile_words,
                                   tile_words)]
            pending[which] = pltpu.async_copy(
                buf.at[pl.ds(0, tile_words)], dst, sems[which])
        for which in (0, 1):
            if pending[which] is not None:
                pending[which].wait()

    return sc_bits


_sc_bits_cached = functools.cache(_make_sc_bits_kernel)

_WEIGHT_SPECS = [
    pl.BlockSpec((_VOCAB, _EMB), lambda j: (0, 0)),
    pl.BlockSpec((_EMB, _HID), lambda j: (0, 0)),
    pl.BlockSpec((1, _HID), lambda j: (0, 0)),
    pl.BlockSpec((_HID, _VOCAB), lambda j: (0, 0)),
    pl.BlockSpec((1, _VOCAB), lambda j: (0, 0)),
]


def kernel(x, emb_table, W1, b1, W2, b2):
    B, L = x.shape
    xT = x.T.reshape(L, B, 1)
    w1t = W1.T
    w2t = W2.T
    b1r = b1.reshape(1, _HID)
    b2r = b2.reshape(1, _VOCAB)
    weights = (emb_table, w1t, b1r, w2t, b2r)

    n_main = L - _NP
    a_out = pl.pallas_call(
        _tc_main_kernel,
        grid=(n_main,),
        in_specs=[
            pl.BlockSpec(memory_space=pltpu.SMEM),
            pl.BlockSpec(memory_space=pltpu.SMEM),
            pl.BlockSpec((1, B, 1), lambda j: (j, 0, 0)),
            *_WEIGHT_SPECS,
        ],
        out_specs=pl.BlockSpec((1, B, 1), lambda j: (j, 0, 0)),
        out_shape=jax.ShapeDtypeStruct((n_main, B, 1), jnp.int32),
    )(jnp.asarray(_KCAT[:, _NP:]), jnp.asarray(_KUNIF[:, _NP:]), xT[_NP:],
      *weights)

    sc_bits = [
        _sc_bits_cached(i)().reshape(_NP, B, _VOCAB) for i in range(_STEPS)
    ]

    xb = xT[:_NP] + a_out[0, 0, 0] * 0

    for i in range(_STEPS):
        xb = pl.pallas_call(
            _tc_consume_kernel,
            grid=(_NP,),
            in_specs=[
                pl.BlockSpec(memory_space=pltpu.SMEM),
                pl.BlockSpec((1, B, 1), lambda j: (j, 0, 0)),
                pl.BlockSpec((1, B, _VOCAB), lambda j: (j, 0, 0)),
                *_WEIGHT_SPECS,
            ],
            out_specs=pl.BlockSpec((1, B, 1), lambda j: (j, 0, 0)),
            out_shape=jax.ShapeDtypeStruct((_NP, B, 1), jnp.int32),
        )(jnp.asarray(_KUNIF[i, :_NP]), xb, sc_bits[i], *weights)

    return jnp.concatenate([xb, a_out], axis=0).reshape(L, B).T
